# Initial kernel scaffold; baseline (speedup 1.0000x reference)
import jax
import jax.numpy as jnp
from jax.experimental import pallas as pl


def kernel(x, text_x, edge_index, Wf, bf, Wt, bt, W1, b1, W2, b2):
    raise NotImplementedError("write your pallas kernel here")


# speedup vs baseline: 17.3271x; 17.3271x over previous
import functools

import jax
import jax.numpy as jnp
from jax import lax
from jax.experimental import pallas as pl
from jax.experimental.pallas import tpu as pltpu
from jax.experimental.pallas import tpu_sc as plsc

N = 10000
E = 320000
D = 128

NC = 2
NS = 16
NW = NC * NS
EPT = E // NW
CW = 80
CH = EPT // CW
NP = 10240
RPT = NP // NS

BN = 2000

_sc_mesh = plsc.VectorSubcoreMesh(core_axis_name="c", subcore_axis_name="s")


@functools.partial(
    pl.kernel,
    mesh=_sc_mesh,
    out_type=jax.ShapeDtypeStruct((NC, NP, D), jnp.float32),
    scratch_types=[
        pltpu.VMEM((CH, CW), jnp.int32),
        pltpu.VMEM((CW, D), jnp.float32),
        pltpu.VMEM_SHARED((NP, D), jnp.float32),
    ],
)
def _deg_kernel(dst_hbm, ones_hbm, zeros_hbm, out_hbm, dst_v, ones_v, acc_sp):
    c = lax.axis_index("c")
    s = lax.axis_index("s")
    wid = c * NS + s
    stripe = pl.ds(s * RPT, RPT)
    pltpu.sync_copy(zeros_hbm.at[stripe], acc_sp.at[stripe])
    pltpu.sync_copy(dst_hbm.at[wid], dst_v)
    pltpu.sync_copy(ones_hbm, ones_v)
    plsc.subcore_barrier()

    def body(j, carry):
        pltpu.sync_copy(ones_v, acc_sp.at[dst_v.at[j]], add=True)
        return carry

    lax.fori_loop(0, CH, body, 0)
    plsc.subcore_barrier()
    pltpu.sync_copy(acc_sp.at[stripe], out_hbm.at[c, stripe])


@functools.partial(
    pl.kernel,
    mesh=_sc_mesh,
    out_type=jax.ShapeDtypeStruct((NC, NP, D), jnp.float32),
    scratch_types=[
        pltpu.VMEM((CH, CW), jnp.int32),
        pltpu.VMEM((CH, CW), jnp.int32),
        pltpu.VMEM((CW, D), jnp.float32),
        pltpu.VMEM_SHARED((NP, D), jnp.float32),
        pltpu.SemaphoreType.DMA,
    ],
)
def _edge_accum_kernel(src_hbm, dst_hbm, hs_hbm, zeros_hbm, out_hbm,
                       src_v, dst_v, rows_v, acc_sp, sem):
    c = lax.axis_index("c")
    s = lax.axis_index("s")
    wid = c * NS + s
    stripe = pl.ds(s * RPT, RPT)
    pltpu.sync_copy(zeros_hbm.at[stripe], acc_sp.at[stripe])
    pltpu.sync_copy(src_hbm.at[wid], src_v)
    pltpu.sync_copy(dst_hbm.at[wid], dst_v)
    plsc.subcore_barrier()

    def body(j, carry):
        pltpu.async_copy(hs_hbm.at[src_v.at[j]], rows_v, sem).wait()
        pltpu.sync_copy(rows_v, acc_sp.at[dst_v.at[j]], add=True)
        return carry

    lax.fori_loop(0, CH, body, 0)
    plsc.subcore_barrier()
    pltpu.sync_copy(acc_sp.at[stripe], out_hbm.at[c, stripe])


def _dinv_of(d0, d1):
    deg = d0[:, 0:1] + d1[:, 0:1] + 1.0
    return lax.rsqrt(deg)


def _tc1_body(x_ref, tx_ref, wf_ref, bf_ref, wt_ref, bt_ref, w1_ref,
              d0_ref, d1_ref, hs_ref):
    h1 = jnp.dot(x_ref[...], wf_ref[...], preferred_element_type=jnp.float32) + bf_ref[...]
    h2 = jnp.dot(tx_ref[...], wt_ref[...], preferred_element_type=jnp.float32) + bt_ref[...]
    hh = (jnp.dot(h1, w1_ref[0:D, :], preferred_element_type=jnp.float32)
          + jnp.dot(h2, w1_ref[D:2 * D, :], preferred_element_type=jnp.float32))
    hs_ref[...] = hh * _dinv_of(d0_ref[...], d1_ref[...])


def _tc2_body(a0_ref, a1_ref, hs1_ref, d0_ref, d1_ref, b1_ref, w2_ref, hs2_ref):
    dinv = _dinv_of(d0_ref[...], d1_ref[...])
    h = dinv * (a0_ref[...] + a1_ref[...] + hs1_ref[...]) + b1_ref[...]
    h = jnp.maximum(h, 0.0)
    hs2_ref[...] = jnp.dot(h, w2_ref[...], preferred_element_type=jnp.float32) * dinv


def _tc3_body(a0_ref, a1_ref, hs2_ref, d0_ref, d1_ref, b2_ref, out_ref):
    dinv = _dinv_of(d0_ref[...], d1_ref[...])
    out_ref[...] = dinv * (a0_ref[...] + a1_ref[...] + hs2_ref[...]) + b2_ref[...]


def _row_spec(width):
    return pl.BlockSpec((BN, width), lambda i: (i, 0))


def _full_spec(shape):
    return pl.BlockSpec(shape, lambda i: tuple(0 for _ in shape))


_GRID = (N // BN,)

_tc1 = pl.pallas_call(
    _tc1_body,
    grid=_GRID,
    in_specs=[
        _row_spec(D), _row_spec(D),
        _full_spec((D, D)), _full_spec((1, D)),
        _full_spec((D, D)), _full_spec((1, D)),
        _full_spec((2 * D, D)),
        _row_spec(D), _row_spec(D),
    ],
    out_specs=_row_spec(D),
    out_shape=jax.ShapeDtypeStruct((N, D), jnp.float32),
)

_tc2 = pl.pallas_call(
    _tc2_body,
    grid=_GRID,
    in_specs=[
        _row_spec(D), _row_spec(D), _row_spec(D),
        _row_spec(D), _row_spec(D),
        _full_spec((1, D)), _full_spec((D, D)),
    ],
    out_specs=_row_spec(D),
    out_shape=jax.ShapeDtypeStruct((N, D), jnp.float32),
)

_tc3 = pl.pallas_call(
    _tc3_body,
    grid=_GRID,
    in_specs=[
        _row_spec(D), _row_spec(D), _row_spec(D),
        _row_spec(D), _row_spec(D),
        _full_spec((1, D)),
    ],
    out_specs=_row_spec(D),
    out_shape=jax.ShapeDtypeStruct((N, D), jnp.float32),
)


def kernel(x, text_x, edge_index, Wf, bf, Wt, bt, W1, b1, W2, b2):
    ei = edge_index.astype(jnp.int32)
    src = ei[0].reshape(NW, CH, CW)
    dst = ei[1].reshape(NW, CH, CW)

    onesD = jnp.ones((CW, D), jnp.float32)
    zerosD = jnp.zeros((NP, D), jnp.float32)
    bf_r = bf.reshape(1, D)
    bt_r = bt.reshape(1, D)
    b1_r = b1.reshape(1, D)
    b2_r = b2.reshape(1, D)

    degp = _deg_kernel(dst, onesD, zerosD)
    d0, d1 = degp[0], degp[1]

    hs1 = _tc1(x, text_x, Wf, bf_r, Wt, bt_r, W1, d0, d1)
    acc1 = _edge_accum_kernel(src, dst, hs1, zerosD)
    hs2 = _tc2(acc1[0], acc1[1], hs1, d0, d1, b1_r, W2)
    acc2 = _edge_accum_kernel(src, dst, hs2, zerosD)
    return _tc3(acc2[0], acc2[1], hs2, d0, d1, b2_r)
